# Initial kernel scaffold; baseline (speedup 1.0000x reference)
import jax
import jax.numpy as jnp
from jax.experimental import pallas as pl


def kernel(batch_user, batch_pos_item, batch_neg_item, user_emb_MF, item_emb_MF, user_emb_MLP, item_emb_MLP, W1, b1, Wf, bf):
    raise NotImplementedError("write your pallas kernel here")


# speedup vs baseline: 2.7581x; 2.7581x over previous
import functools

import jax
import jax.numpy as jnp
from jax import lax
from jax.experimental import pallas as pl
from jax.experimental.pallas import tpu as pltpu
from jax.experimental.pallas import tpu_sc as plsc

B = 16384
D = 128
NC, NS = 2, 16
NW = NC * NS
BPW = B // NW
CH = 128
NCK = BPW // CH


def _gather6(u3, p3, n3, umf, imf, umlp, imlp):
    mesh = plsc.VectorSubcoreMesh(core_axis_name="c", subcore_axis_name="s")

    @functools.partial(
        pl.kernel,
        mesh=mesh,
        out_type=[jax.ShapeDtypeStruct((B, D), jnp.float32)] * 6,
        scratch_types=[
            pltpu.VMEM((NCK, CH), jnp.int32),
            pltpu.VMEM((NCK, CH), jnp.int32),
            pltpu.VMEM((NCK, CH), jnp.int32),
            pltpu.VMEM((CH, D), jnp.float32),
            pltpu.VMEM((CH, D), jnp.float32),
            pltpu.SemaphoreType.DMA,
            pltpu.SemaphoreType.DMA,
            pltpu.SemaphoreType.DMA,
            pltpu.SemaphoreType.DMA,
        ],
    )
    def k(u_idx_h, p_idx_h, n_idx_h, umf_h, imf_h, umlp_h, imlp_h,
          o_umf, o_imf_p, o_imf_n, o_umlp, o_imlp_p, o_imlp_n,
          xu, xp, xn, buf0, buf1, sg0, sg1, sw0, sw1):
        wid = lax.axis_index("s") * NC + lax.axis_index("c")
        pltpu.sync_copy(u_idx_h.at[wid], xu)
        pltpu.sync_copy(p_idx_h.at[wid], xp)
        pltpu.sync_copy(n_idx_h.at[wid], xn)
        base = wid * BPW
        bufs = (buf0, buf1)
        sgs = (sg0, sg1)
        sws = (sw0, sw1)
        steps = []
        for tbl, xi, out in ((umf_h, xu, o_umf), (imf_h, xp, o_imf_p),
                             (imf_h, xn, o_imf_n), (umlp_h, xu, o_umlp),
                             (imlp_h, xp, o_imlp_p), (imlp_h, xn, o_imlp_n)):
            for c in range(NCK):
                steps.append((tbl, xi, c, out))
        nst = len(steps)

        def start_gather(j):
            tbl, xi, c, _ = steps[j]
            return pltpu.async_copy(tbl.at[xi.at[c]], bufs[j % 2], sgs[j % 2])

        gh = [None] * nst
        wh = [None] * nst
        gh[0] = start_gather(0)
        for j in range(nst):
            b = j % 2
            _, _, c, out = steps[j]
            if j + 1 < nst:
                if j >= 1:
                    wh[j - 1].wait()
                gh[j + 1] = start_gather(j + 1)
            gh[j].wait()
            wh[j] = pltpu.async_copy(
                bufs[b], out.at[pl.ds(base + c * CH, CH)], sws[b])
        wh[nst - 2].wait()
        wh[nst - 1].wait()

    return k(u3, p3, n3, umf, imf, umlp, imlp)


def _dense(umlp_g, imlp_p_g, imlp_n_g, umf_g, imf_p_g, imf_n_g,
           w1a, w1b, b1r, wf1, wf2, bfv):
    BB = 2048

    def body(umlp_r, imlp_p_r, imlp_n_r, umf_r, imf_p_r, imf_n_r,
             w1a_r, w1b_r, b1_r, wf1_r, wf2_r, bf_r, pos_r, neg_r):
        hu = jnp.dot(umlp_r[...], w1a_r[...],
                     preferred_element_type=jnp.float32)
        hp = jnp.maximum(
            hu + jnp.dot(imlp_p_r[...], w1b_r[...],
                         preferred_element_type=jnp.float32) + b1_r[...], 0.0)
        hn = jnp.maximum(
            hu + jnp.dot(imlp_n_r[...], w1b_r[...],
                         preferred_element_type=jnp.float32) + b1_r[...], 0.0)
        u = umf_r[...]
        bf0 = bf_r[0, 0]
        pos_r[...] = (jnp.sum(u * imf_p_r[...] * wf1_r[...], axis=1,
                              keepdims=True)
                      + jnp.sum(hp * wf2_r[...], axis=1, keepdims=True) + bf0)
        neg_r[...] = (jnp.sum(u * imf_n_r[...] * wf1_r[...], axis=1,
                              keepdims=True)
                      + jnp.sum(hn * wf2_r[...], axis=1, keepdims=True) + bf0)

    bspec_in = pl.BlockSpec((BB, D), lambda i: (i, 0))
    bspec_w = pl.BlockSpec((D, D), lambda i: (0, 0))
    bspec_r = pl.BlockSpec((1, D), lambda i: (0, 0))
    bspec_bf = pl.BlockSpec((1, 1), lambda i: (0, 0))
    bspec_out = pl.BlockSpec((BB, 1), lambda i: (i, 0))
    return pl.pallas_call(
        body,
        grid=(B // BB,),
        in_specs=[bspec_in] * 6 + [bspec_w, bspec_w, bspec_r, bspec_r,
                                   bspec_r, bspec_bf],
        out_specs=[bspec_out, bspec_out],
        out_shape=[jax.ShapeDtypeStruct((B, 1), jnp.float32)] * 2,
    )(umlp_g, imlp_p_g, imlp_n_g, umf_g, imf_p_g, imf_n_g,
      w1a, w1b, b1r, wf1, wf2, bfv)


def kernel(batch_user, batch_pos_item, batch_neg_item,
           user_emb_MF, item_emb_MF, user_emb_MLP, item_emb_MLP,
           W1, b1, Wf, bf):
    u3 = batch_user.astype(jnp.int32).reshape(NW, NCK, CH)
    p3 = batch_pos_item.astype(jnp.int32).reshape(NW, NCK, CH)
    n3 = batch_neg_item.astype(jnp.int32).reshape(NW, NCK, CH)
    o_umf, o_imf_p, o_imf_n, o_umlp, o_imlp_p, o_imlp_n = _gather6(
        u3, p3, n3, user_emb_MF, item_emb_MF, user_emb_MLP, item_emb_MLP)
    w1a = W1[:D]
    w1b = W1[D:]
    b1r = b1.reshape(1, D)
    wf1 = Wf[:D, 0].reshape(1, D)
    wf2 = Wf[D:, 0].reshape(1, D)
    bfv = bf.reshape(1, 1)
    pos, neg = _dense(o_umlp, o_imlp_p, o_imlp_n, o_umf, o_imf_p, o_imf_n,
                      w1a, w1b, b1r, wf1, wf2, bfv)
    return (pos, neg)
